# baseline (speedup 1.0000x reference)
import functools

import jax
import jax.numpy as jnp
from jax import lax
from jax.experimental import pallas as pl
from jax.experimental.pallas import tpu as pltpu
from jax.experimental.pallas import tpu_sc as plsc

N_Q = 1024
N_T = 100000
D_F = 32
BK = 1024
NB = (N_T + BK - 1) // BK
BIG = float("inf")
IBIG = 2**31 - 1


def _insert(v0, i0, v1, i1, v2, i2, m, j):
    lt0 = m < v0
    lt1 = m < v1
    lt2 = m < v2
    nv0 = jnp.where(lt0, m, v0)
    ni0 = jnp.where(lt0, j, i0)
    nv1 = jnp.where(lt0, v0, jnp.where(lt1, m, v1))
    ni1 = jnp.where(lt0, i0, jnp.where(lt1, j, i1))
    nv2 = jnp.where(lt1, v1, jnp.where(lt2, m, v2))
    ni2 = jnp.where(lt1, i1, jnp.where(lt2, j, i2))
    return nv0, ni0, nv1, ni1, nv2, ni2


def _topk_body(sf_ref, tf_ref, w_out, idx_out,
               v0, i0, v1, i1, v2, i2):
    i = pl.program_id(0)

    @pl.when(i == 0)
    def _init():
        for r in (v0, v1, v2):
            r[...] = jnp.full((N_Q, 1), BIG, jnp.float32)
        for r in (i0, i1, i2):
            r[...] = jnp.full((N_Q, 1), IBIG, jnp.int32)

    sf = sf_ref[...]
    tf = tf_ref[...]
    dp = jax.lax.dot_general(
        sf, tf, (((1,), (1,)), ((), ())),
        preferred_element_type=jnp.float32)
    ss = jnp.sum(sf * sf, axis=1, keepdims=True)
    tsq = jnp.sum(tf * tf, axis=1)[None, :]
    d = ss - 2.0 * dp + tsq

    gidx = i * BK + lax.broadcasted_iota(jnp.int32, (N_Q, BK), 1)

    d = jnp.where(gidx < N_T, d, BIG)

    a0, b0, a1, b1, a2, b2 = (v0[...], i0[...], v1[...], i1[...],
                              v2[...], i2[...])
    for r in range(3):
        m = jnp.min(d, axis=1, keepdims=True)
        j = jnp.min(jnp.where(d == m, gidx, IBIG), axis=1,
                    keepdims=True)
        if r < 2:
            d = jnp.where(gidx == j, BIG, d)
        a0, b0, a1, b1, a2, b2 = _insert(a0, b0, a1, b1, a2, b2, m, j)
    v0[...], i0[...], v1[...], i1[...], v2[...], i2[...] = (
        a0, b0, a1, b1, a2, b2)

    @pl.when(i == NB - 1)
    def _finalize():
        e0 = jnp.ones_like(a0)
        e1 = jnp.exp(a0 - a1)
        e2 = jnp.exp(a0 - a2)
        tot = e0 + e1 + e2
        zf = jnp.zeros((N_Q, 5), jnp.float32)
        w_out[...] = jnp.concatenate(
            [e0 / tot, e1 / tot, e2 / tot, zf], axis=1)
        zi = jnp.zeros((N_Q, 5), jnp.int32)
        idx_out[...] = jnp.concatenate([b0, b1, b2, zi], axis=1)


@jax.jit
def _topk(source_feats, target_feats):
    w, idx = pl.pallas_call(
        _topk_body,
        grid=(NB,),
        in_specs=[
            pl.BlockSpec((N_Q, D_F), lambda i: (0, 0)),
            pl.BlockSpec((BK, D_F), lambda i: (i, 0)),
        ],
        out_specs=[
            pl.BlockSpec((N_Q, 8), lambda i: (0, 0)),
            pl.BlockSpec((N_Q, 8), lambda i: (0, 0)),
        ],
        out_shape=[
            jax.ShapeDtypeStruct((N_Q, 8), jnp.float32),
            jax.ShapeDtypeStruct((N_Q, 8), jnp.int32),
        ],
        scratch_shapes=[pltpu.VMEM((N_Q, 1), jnp.float32),
                        pltpu.VMEM((N_Q, 1), jnp.int32)] * 3,
        compiler_params=pltpu.CompilerParams(
            dimension_semantics=("arbitrary",)),
    )(source_feats, target_feats)
    return w[:, :3], idx[:, :3]


_NW = 32
_QW = N_Q // _NW
_CW = 3 * _QW
_PD = 16


def _sc_body(idx_hbm, w_hbm, table_hbm, out_hbm, idx_v, w_v, rows_v,
             out_v, sem):
    wid = lax.axis_index("s") * 2 + lax.axis_index("c")
    base = wid * _CW
    pltpu.sync_copy(idx_hbm.at[pl.ds(base, _CW)], idx_v)
    pltpu.sync_copy(w_hbm.at[pl.ds(base, _CW), :], w_v)
    pltpu.async_copy(table_hbm.at[idx_v], rows_v, sem).wait()
    for q in range(_QW):
        acc = (w_v[3 * q, :] * rows_v[3 * q, :]
               + w_v[3 * q + 1, :] * rows_v[3 * q + 1, :]
               + w_v[3 * q + 2, :] * rows_v[3 * q + 2, :])
        out_v[q, :] = acc
    pltpu.sync_copy(out_v, out_hbm.at[pl.ds(wid * _QW, _QW), :])


@jax.jit
def _sc_combine(idx_flat, w_rows, table_pad):
    mesh = plsc.VectorSubcoreMesh(core_axis_name="c", subcore_axis_name="s")
    run = functools.partial(
        pl.kernel,
        mesh=mesh,
        compiler_params=pltpu.CompilerParams(use_tc_tiling_on_sc=False),
        out_type=jax.ShapeDtypeStruct((N_Q, _PD), jnp.float32),
        scratch_types=[
            pltpu.VMEM((_CW,), jnp.int32),
            pltpu.VMEM((_CW, _PD), jnp.float32),
            pltpu.VMEM((_CW, _PD), jnp.float32),
            pltpu.VMEM((_QW, _PD), jnp.float32),
            pltpu.SemaphoreType.DMA,
        ],
    )(_sc_body)
    return run(idx_flat, w_rows, table_pad)


def kernel(source_feats, target_feats, target_points):
    w, idx = _topk(source_feats, target_feats)
    table_pad = jnp.pad(target_points, ((0, 0), (0, _PD - 3)))
    w_rows = jnp.broadcast_to(w.reshape(-1)[:, None], (3 * N_Q, _PD))
    out = _sc_combine(idx.reshape(-1), w_rows, table_pad)
    return out[:, :3]


# speedup vs baseline: 1.3574x; 1.3574x over previous
import functools

import jax
import jax.numpy as jnp
from jax import lax
from jax.experimental import pallas as pl
from jax.experimental.pallas import tpu as pltpu
from jax.experimental.pallas import tpu_sc as plsc

N_Q = 1024
N_T = 100000
D_F = 32
BK = 4096
NB = (N_T + BK - 1) // BK
BIG = float("inf")
IBIG = 2**31 - 1


def _insert(v0, i0, v1, i1, v2, i2, m, j):
    lt0 = m < v0
    lt1 = m < v1
    lt2 = m < v2
    nv0 = jnp.where(lt0, m, v0)
    ni0 = jnp.where(lt0, j, i0)
    nv1 = jnp.where(lt0, v0, jnp.where(lt1, m, v1))
    ni1 = jnp.where(lt0, i0, jnp.where(lt1, j, i1))
    nv2 = jnp.where(lt1, v1, jnp.where(lt2, m, v2))
    ni2 = jnp.where(lt1, i1, jnp.where(lt2, j, i2))
    return nv0, ni0, nv1, ni1, nv2, ni2


def _topk_body(sf_ref, tf_ref, w_out, idx_out,
               v0, i0, v1, i1, v2, i2):
    i = pl.program_id(0)

    @pl.when(i == 0)
    def _init():
        for r in (v0, v1, v2):
            r[...] = jnp.full((N_Q, 1), BIG, jnp.float32)
        for r in (i0, i1, i2):
            r[...] = jnp.full((N_Q, 1), IBIG, jnp.int32)

    sf = sf_ref[...]
    tf = tf_ref[...]
    dp2 = jax.lax.dot_general(
        -2.0 * sf, tf, (((1,), (1,)), ((), ())),
        preferred_element_type=jnp.float32)
    ss = jnp.sum(sf * sf, axis=1, keepdims=True)
    lane_row = lax.broadcasted_iota(jnp.int32, (1, BK), 1)
    tsq = jnp.sum(tf * tf, axis=1)[None, :]
    tsq = jnp.where(i * BK + lane_row < N_T, tsq, BIG)
    d = (ss + dp2) + tsq

    lane_f = lax.broadcasted_iota(jnp.int32, (N_Q, BK), 1).astype(jnp.float32)

    a0, b0, a1, b1, a2, b2 = (v0[...], i0[...], v1[...], i1[...],
                              v2[...], i2[...])
    for r in range(3):
        m = jnp.min(d, axis=1, keepdims=True)
        jl = jnp.min(jnp.where(d == m, lane_f, BIG), axis=1,
                     keepdims=True)
        if r < 2:
            d = jnp.where(lane_f == jl, BIG, d)
        j = i * BK + jl.astype(jnp.int32)
        a0, b0, a1, b1, a2, b2 = _insert(a0, b0, a1, b1, a2, b2, m, j)
    v0[...], i0[...], v1[...], i1[...], v2[...], i2[...] = (
        a0, b0, a1, b1, a2, b2)

    @pl.when(i == NB - 1)
    def _finalize():
        e0 = jnp.ones_like(a0)
        e1 = jnp.exp(a0 - a1)
        e2 = jnp.exp(a0 - a2)
        tot = e0 + e1 + e2
        zf = jnp.zeros((N_Q, 5), jnp.float32)
        w_out[...] = jnp.concatenate(
            [e0 / tot, e1 / tot, e2 / tot, zf], axis=1)
        zi = jnp.zeros((N_Q, 5), jnp.int32)
        idx_out[...] = jnp.concatenate([b0, b1, b2, zi], axis=1)


@jax.jit
def _topk(source_feats, target_feats):
    w, idx = pl.pallas_call(
        _topk_body,
        grid=(NB,),
        in_specs=[
            pl.BlockSpec((N_Q, D_F), lambda i: (0, 0)),
            pl.BlockSpec((BK, D_F), lambda i: (i, 0)),
        ],
        out_specs=[
            pl.BlockSpec((N_Q, 8), lambda i: (0, 0)),
            pl.BlockSpec((N_Q, 8), lambda i: (0, 0)),
        ],
        out_shape=[
            jax.ShapeDtypeStruct((N_Q, 8), jnp.float32),
            jax.ShapeDtypeStruct((N_Q, 8), jnp.int32),
        ],
        scratch_shapes=[pltpu.VMEM((N_Q, 1), jnp.float32),
                        pltpu.VMEM((N_Q, 1), jnp.int32)] * 3,
        compiler_params=pltpu.CompilerParams(
            dimension_semantics=("arbitrary",)),
    )(source_feats, target_feats)
    return w[:, :3], idx[:, :3]


_NW = 32
_QW = N_Q // _NW
_CW = 3 * _QW
_PD = 16


def _sc_body(idx_hbm, w_hbm, table_hbm, out_hbm, idx_v, w_v, rows_v,
             out_v, sem):
    wid = lax.axis_index("s") * 2 + lax.axis_index("c")
    base = wid * _CW
    pltpu.sync_copy(idx_hbm.at[pl.ds(base, _CW)], idx_v)
    pltpu.sync_copy(w_hbm.at[pl.ds(base, _CW), :], w_v)
    pltpu.async_copy(table_hbm.at[idx_v], rows_v, sem).wait()
    for q in range(_QW):
        acc = (w_v[3 * q, :] * rows_v[3 * q, :]
               + w_v[3 * q + 1, :] * rows_v[3 * q + 1, :]
               + w_v[3 * q + 2, :] * rows_v[3 * q + 2, :])
        out_v[q, :] = acc
    pltpu.sync_copy(out_v, out_hbm.at[pl.ds(wid * _QW, _QW), :])


@jax.jit
def _sc_combine(idx_flat, w_rows, table_pad):
    mesh = plsc.VectorSubcoreMesh(core_axis_name="c", subcore_axis_name="s")
    run = functools.partial(
        pl.kernel,
        mesh=mesh,
        compiler_params=pltpu.CompilerParams(use_tc_tiling_on_sc=False),
        out_type=jax.ShapeDtypeStruct((N_Q, _PD), jnp.float32),
        scratch_types=[
            pltpu.VMEM((_CW,), jnp.int32),
            pltpu.VMEM((_CW, _PD), jnp.float32),
            pltpu.VMEM((_CW, _PD), jnp.float32),
            pltpu.VMEM((_QW, _PD), jnp.float32),
            pltpu.SemaphoreType.DMA,
        ],
    )(_sc_body)
    return run(idx_flat, w_rows, table_pad)


def kernel(source_feats, target_feats, target_points):
    w, idx = _topk(source_feats, target_feats)
    table_pad = jnp.pad(target_points, ((0, 0), (0, _PD - 3)))
    w_rows = jnp.broadcast_to(w.reshape(-1)[:, None], (3 * N_Q, _PD))
    out = _sc_combine(idx.reshape(-1), w_rows, table_pad)
    return out[:, :3]


# speedup vs baseline: 1.3587x; 1.0009x over previous
import functools

import jax
import jax.numpy as jnp
from jax import lax
from jax.experimental import pallas as pl
from jax.experimental.pallas import tpu as pltpu
from jax.experimental.pallas import tpu_sc as plsc

N_Q = 1024
N_T = 100000
D_F = 32
BK = 4096
NB = (N_T + BK - 1) // BK
BIG = float("inf")
IBIG = 2**31 - 1


def _insert(v0, i0, v1, i1, v2, i2, m, j):
    lt0 = m < v0
    lt1 = m < v1
    lt2 = m < v2
    nv0 = jnp.where(lt0, m, v0)
    ni0 = jnp.where(lt0, j, i0)
    nv1 = jnp.where(lt0, v0, jnp.where(lt1, m, v1))
    ni1 = jnp.where(lt0, i0, jnp.where(lt1, j, i1))
    nv2 = jnp.where(lt1, v1, jnp.where(lt2, m, v2))
    ni2 = jnp.where(lt1, i1, jnp.where(lt2, j, i2))
    return nv0, ni0, nv1, ni1, nv2, ni2


def _topk_body(sf_ref, tf_ref, w_out, idx_out,
               v0, i0, v1, i1, v2, i2):
    i = pl.program_id(0)

    @pl.when(i == 0)
    def _init():
        for r in (v0, v1, v2):
            r[...] = jnp.full((N_Q, 1), BIG, jnp.float32)
        for r in (i0, i1, i2):
            r[...] = jnp.full((N_Q, 1), IBIG, jnp.int32)

    sf = sf_ref[...]
    tf = tf_ref[...]
    dp2 = jax.lax.dot_general(
        -2.0 * sf, tf, (((1,), (1,)), ((), ())),
        preferred_element_type=jnp.float32)
    ss = jnp.sum(sf * sf, axis=1, keepdims=True)
    lane_row = lax.broadcasted_iota(jnp.int32, (1, BK), 1)
    tsq = jnp.sum(tf * tf, axis=1)[None, :]
    tsq = jnp.where(i * BK + lane_row < N_T, tsq, BIG)
    d = (ss + dp2) + tsq

    lane_f = lax.broadcasted_iota(jnp.int32, (N_Q, BK), 1).astype(jnp.float32)

    a0, b0, a1, b1, a2, b2 = (v0[...], i0[...], v1[...], i1[...],
                              v2[...], i2[...])
    for r in range(3):
        m = jnp.min(d, axis=1, keepdims=True)
        jl = jnp.min(jnp.where(d == m, lane_f, BIG), axis=1,
                     keepdims=True)
        if r < 2:
            d = jnp.where(lane_f == jl, BIG, d)
        j = i * BK + jl.astype(jnp.int32)
        a0, b0, a1, b1, a2, b2 = _insert(a0, b0, a1, b1, a2, b2, m, j)
    v0[...], i0[...], v1[...], i1[...], v2[...], i2[...] = (
        a0, b0, a1, b1, a2, b2)

    @pl.when(i == NB - 1)
    def _finalize():
        e0 = jnp.ones_like(a0)
        e1 = jnp.exp(a0 - a1)
        e2 = jnp.exp(a0 - a2)
        tot = e0 + e1 + e2
        zf = jnp.zeros((N_Q, 5), jnp.float32)
        w_out[...] = jnp.concatenate(
            [e0 / tot, e1 / tot, e2 / tot, zf], axis=1)
        zi = jnp.zeros((N_Q, 5), jnp.int32)
        idx_out[...] = jnp.concatenate([b0, b1, b2, zi], axis=1)


def _topk(source_feats, target_feats):
    w, idx = pl.pallas_call(
        _topk_body,
        grid=(NB,),
        in_specs=[
            pl.BlockSpec((N_Q, D_F), lambda i: (0, 0)),
            pl.BlockSpec((BK, D_F), lambda i: (i, 0)),
        ],
        out_specs=[
            pl.BlockSpec((N_Q, 8), lambda i: (0, 0)),
            pl.BlockSpec((N_Q, 8), lambda i: (0, 0)),
        ],
        out_shape=[
            jax.ShapeDtypeStruct((N_Q, 8), jnp.float32),
            jax.ShapeDtypeStruct((N_Q, 8), jnp.int32),
        ],
        scratch_shapes=[pltpu.VMEM((N_Q, 1), jnp.float32),
                        pltpu.VMEM((N_Q, 1), jnp.int32)] * 3,
        compiler_params=pltpu.CompilerParams(
            dimension_semantics=("arbitrary",)),
    )(source_feats, target_feats)
    return w[:, :3], idx[:, :3]


_NW = 32
_QW = N_Q // _NW
_CW = 3 * _QW
_PD = 16


def _sc_body(idx_hbm, w_hbm, table_hbm, out_hbm, idx_v, w_v, rows_v,
             out_v, sem):
    wid = lax.axis_index("s") * 2 + lax.axis_index("c")
    base = wid * _CW
    pltpu.sync_copy(idx_hbm.at[pl.ds(base, _CW)], idx_v)
    pltpu.sync_copy(w_hbm.at[pl.ds(base, _CW), :], w_v)
    pltpu.async_copy(table_hbm.at[idx_v], rows_v, sem).wait()
    for q in range(_QW):
        acc = (w_v[3 * q, :] * rows_v[3 * q, :]
               + w_v[3 * q + 1, :] * rows_v[3 * q + 1, :]
               + w_v[3 * q + 2, :] * rows_v[3 * q + 2, :])
        out_v[q, :] = acc
    pltpu.sync_copy(out_v, out_hbm.at[pl.ds(wid * _QW, _QW), :])


def _sc_combine(idx_flat, w_rows, table_pad):
    mesh = plsc.VectorSubcoreMesh(core_axis_name="c", subcore_axis_name="s")
    run = functools.partial(
        pl.kernel,
        mesh=mesh,
        compiler_params=pltpu.CompilerParams(use_tc_tiling_on_sc=False),
        out_type=jax.ShapeDtypeStruct((N_Q, _PD), jnp.float32),
        scratch_types=[
            pltpu.VMEM((_CW,), jnp.int32),
            pltpu.VMEM((_CW, _PD), jnp.float32),
            pltpu.VMEM((_CW, _PD), jnp.float32),
            pltpu.VMEM((_QW, _PD), jnp.float32),
            pltpu.SemaphoreType.DMA,
        ],
    )(_sc_body)
    return run(idx_flat, w_rows, table_pad)


@jax.jit
def kernel(source_feats, target_feats, target_points):
    w, idx = _topk(source_feats, target_feats)
    table_pad = jnp.pad(target_points, ((0, 0), (0, _PD - 3)))
    w_rows = jnp.broadcast_to(w.reshape(-1)[:, None], (3 * N_Q, _PD))
    out = _sc_combine(idx.reshape(-1), w_rows, table_pad)
    return out[:, :3]
